# baseline (device time: 18582 ns/iter reference)
import jax
import jax.numpy as jnp
from jax import lax
from jax.experimental import pallas as pl
from jax.experimental.pallas import tpu as pltpu

M = 1024
N = 1024
HALF = 512
C = 16
R = HALF // C


def kernel(x):
    def body(
        x_ref, out_ref,
        xstage, ysend, yrecv, acc,
        sem_local, sem_ys, sem_yr, sem_xs, sem_xr,
    ):
        my_x = lax.axis_index("x")
        my_y = lax.axis_index("y")
        row0 = my_x * HALF
        y_peer = (my_x, 1 - my_y)
        x_peer = (1 - my_x, my_y)

        local = []
        for i in range(C):
            cp = pltpu.make_async_copy(
                x_ref.at[0, pl.ds(row0 + i * R, R), :],
                xstage.at[i],
                sem_local.at[i],
            )
            cp.start()
            local.append(cp)

        barrier_sem = pltpu.get_barrier_semaphore()
        pl.semaphore_signal(
            barrier_sem, inc=1,
            device_id=y_peer, device_id_type=pl.DeviceIdType.MESH,
        )
        pl.semaphore_signal(
            barrier_sem, inc=1,
            device_id=x_peer, device_id_type=pl.DeviceIdType.MESH,
        )
        pl.semaphore_wait(barrier_sem, 2)

        rdmas_y = []
        for i in range(C):
            local[i].wait()
            v = xstage[i].astype(jnp.bfloat16)

            @pl.when(my_y == 0)
            def _(i=i, v=v):
                ysend[i] = v[:, HALF:]
                acc[i] = v[:, :HALF]

            @pl.when(my_y == 1)
            def _(i=i, v=v):
                ysend[i] = v[:, :HALF]
                acc[i] = v[:, HALF:]

            r = pltpu.make_async_remote_copy(
                src_ref=ysend.at[i],
                dst_ref=yrecv.at[i],
                send_sem=sem_ys.at[i],
                recv_sem=sem_yr.at[i],
                device_id=y_peer,
                device_id_type=pl.DeviceIdType.MESH,
            )
            r.start()
            rdmas_y.append(r)

        rdmas_x = []
        for i in range(C):
            rdmas_y[i].wait_recv()
            out_ref[pl.ds(row0 + i * R, R), :] = acc[i] + yrecv[i]
            r = pltpu.make_async_remote_copy(
                src_ref=out_ref.at[pl.ds(row0 + i * R, R), :],
                dst_ref=out_ref.at[pl.ds(row0 + i * R, R), :],
                send_sem=sem_xs.at[i],
                recv_sem=sem_xr.at[i],
                device_id=x_peer,
                device_id_type=pl.DeviceIdType.MESH,
            )
            r.start()
            rdmas_x.append(r)

        for i in range(C):
            rdmas_y[i].wait_send()
            rdmas_x[i].wait_send()
            rdmas_x[i].wait_recv()

    return pl.pallas_call(
        body,
        out_shape=jax.ShapeDtypeStruct((M, HALF), jnp.bfloat16),
        in_specs=[pl.BlockSpec(memory_space=pl.ANY)],
        out_specs=pl.BlockSpec(memory_space=pltpu.VMEM),
        scratch_shapes=[
            pltpu.VMEM((C, R, N), jnp.float32),
            pltpu.VMEM((C, R, HALF), jnp.bfloat16),
            pltpu.VMEM((C, R, HALF), jnp.bfloat16),
            pltpu.VMEM((C, R, HALF), jnp.bfloat16),
            pltpu.SemaphoreType.DMA((C,)),
            pltpu.SemaphoreType.DMA((C,)),
            pltpu.SemaphoreType.DMA((C,)),
            pltpu.SemaphoreType.DMA((C,)),
            pltpu.SemaphoreType.DMA((C,)),
        ],
        compiler_params=pltpu.CompilerParams(collective_id=0),
    )(x)


# device time: 16401 ns/iter; 1.1330x vs baseline; 1.1330x over previous
import jax
import jax.numpy as jnp
from jax import lax
from jax.experimental import pallas as pl
from jax.experimental.pallas import tpu as pltpu

M = 1024
N = 1024
HALF = 512

SIZES = (32, 32, 64, 64, 64, 64, 64, 64, 64)
OFFS = tuple(sum(SIZES[:i]) for i in range(len(SIZES)))
C = len(SIZES)
assert sum(SIZES) == HALF


def kernel(x):
    def body(
        x_ref, out_ref,
        xstage, ysend, yrecv, acc,
        sem_local, sem_ys, sem_yr, sem_xs, sem_xr,
    ):
        my_x = lax.axis_index("x")
        my_y = lax.axis_index("y")
        row0 = my_x * HALF
        y_peer = (my_x, 1 - my_y)
        x_peer = (1 - my_x, my_y)

        local = []
        for i in range(C):
            cp = pltpu.make_async_copy(
                x_ref.at[0, pl.ds(row0 + OFFS[i], SIZES[i]), :],
                xstage.at[pl.ds(OFFS[i], SIZES[i])],
                sem_local.at[i],
            )
            cp.start()
            local.append(cp)

        barrier_sem = pltpu.get_barrier_semaphore()
        pl.semaphore_signal(
            barrier_sem, inc=1,
            device_id=y_peer, device_id_type=pl.DeviceIdType.MESH,
        )
        pl.semaphore_signal(
            barrier_sem, inc=1,
            device_id=x_peer, device_id_type=pl.DeviceIdType.MESH,
        )

        def stage(i):
            local[i].wait()
            o, s = OFFS[i], SIZES[i]
            v = xstage[o:o + s].astype(jnp.bfloat16)

            @pl.when(my_y == 0)
            def _():
                ysend[o:o + s] = v[:, HALF:]
                acc[o:o + s] = v[:, :HALF]

            @pl.when(my_y == 1)
            def _():
                ysend[o:o + s] = v[:, :HALF]
                acc[o:o + s] = v[:, HALF:]

        stage(0)
        pl.semaphore_wait(barrier_sem, 2)

        rdmas_y = []
        for i in range(C):
            if i > 0:
                stage(i)
            r = pltpu.make_async_remote_copy(
                src_ref=ysend.at[pl.ds(OFFS[i], SIZES[i])],
                dst_ref=yrecv.at[pl.ds(OFFS[i], SIZES[i])],
                send_sem=sem_ys.at[i],
                recv_sem=sem_yr.at[i],
                device_id=y_peer,
                device_id_type=pl.DeviceIdType.MESH,
            )
            r.start()
            rdmas_y.append(r)

        rdmas_x = []
        for i in range(C):
            o, s = OFFS[i], SIZES[i]
            rdmas_y[i].wait_recv()
            out_ref[pl.ds(row0 + o, s), :] = acc[o:o + s] + yrecv[o:o + s]
            r = pltpu.make_async_remote_copy(
                src_ref=out_ref.at[pl.ds(row0 + o, s), :],
                dst_ref=out_ref.at[pl.ds(row0 + o, s), :],
                send_sem=sem_xs.at[i],
                recv_sem=sem_xr.at[i],
                device_id=x_peer,
                device_id_type=pl.DeviceIdType.MESH,
            )
            r.start()
            rdmas_x.append(r)

        for i in range(C):
            rdmas_y[i].wait_send()
            rdmas_x[i].wait_send()
            rdmas_x[i].wait_recv()

    return pl.pallas_call(
        body,
        out_shape=jax.ShapeDtypeStruct((M, HALF), jnp.bfloat16),
        in_specs=[pl.BlockSpec(memory_space=pl.ANY)],
        out_specs=pl.BlockSpec(memory_space=pltpu.VMEM),
        scratch_shapes=[
            pltpu.VMEM((HALF, N), jnp.float32),
            pltpu.VMEM((HALF, HALF), jnp.bfloat16),
            pltpu.VMEM((HALF, HALF), jnp.bfloat16),
            pltpu.VMEM((HALF, HALF), jnp.bfloat16),
            pltpu.SemaphoreType.DMA((C,)),
            pltpu.SemaphoreType.DMA((C,)),
            pltpu.SemaphoreType.DMA((C,)),
            pltpu.SemaphoreType.DMA((C,)),
            pltpu.SemaphoreType.DMA((C,)),
        ],
        compiler_params=pltpu.CompilerParams(collective_id=0),
    )(x)


# device time: 16040 ns/iter; 1.1585x vs baseline; 1.0225x over previous
import jax
import jax.numpy as jnp
from jax import lax
from jax.experimental import pallas as pl
from jax.experimental.pallas import tpu as pltpu

M = 1024
N = 1024
HALF = 512

SIZES = (64, 64, 64, 64, 64, 64, 64, 64)
OFFS = tuple(sum(SIZES[:i]) for i in range(len(SIZES)))
C = len(SIZES)
assert sum(SIZES) == HALF


def kernel(x):
    def body(
        x_ref, out_ref,
        xstage, ysend, yrecv, acc,
        sem_local, sem_ys, sem_yr, sem_xs, sem_xr,
    ):
        my_x = lax.axis_index("x")
        my_y = lax.axis_index("y")
        row0 = my_x * HALF
        y_peer = (my_x, 1 - my_y)
        x_peer = (1 - my_x, my_y)

        local = []
        for i in range(C):
            cp = pltpu.make_async_copy(
                x_ref.at[0, pl.ds(row0 + OFFS[i], SIZES[i]), :],
                xstage.at[pl.ds(OFFS[i], SIZES[i])],
                sem_local.at[i],
            )
            cp.start()
            local.append(cp)

        barrier_sem = pltpu.get_barrier_semaphore()
        pl.semaphore_signal(
            barrier_sem, inc=1,
            device_id=y_peer, device_id_type=pl.DeviceIdType.MESH,
        )
        pl.semaphore_signal(
            barrier_sem, inc=1,
            device_id=x_peer, device_id_type=pl.DeviceIdType.MESH,
        )

        def stage(i):
            local[i].wait()
            o, s = OFFS[i], SIZES[i]
            v = xstage[o:o + s].astype(jnp.bfloat16)

            @pl.when(my_y == 0)
            def _():
                ysend[o:o + s] = v[:, HALF:]
                acc[o:o + s] = v[:, :HALF]

            @pl.when(my_y == 1)
            def _():
                ysend[o:o + s] = v[:, :HALF]
                acc[o:o + s] = v[:, HALF:]

        stage(0)
        pl.semaphore_wait(barrier_sem, 2)

        rdmas_y = []
        for i in range(C):
            if i > 0:
                stage(i)
            r = pltpu.make_async_remote_copy(
                src_ref=ysend.at[pl.ds(OFFS[i], SIZES[i])],
                dst_ref=yrecv.at[pl.ds(OFFS[i], SIZES[i])],
                send_sem=sem_ys.at[i],
                recv_sem=sem_yr.at[i],
                device_id=y_peer,
                device_id_type=pl.DeviceIdType.MESH,
            )
            r.start()
            rdmas_y.append(r)

        rdmas_x = []
        for i in range(C):
            o, s = OFFS[i], SIZES[i]
            rdmas_y[i].wait_recv()
            out_ref[pl.ds(row0 + o, s), :] = acc[o:o + s] + yrecv[o:o + s]
            r = pltpu.make_async_remote_copy(
                src_ref=out_ref.at[pl.ds(row0 + o, s), :],
                dst_ref=out_ref.at[pl.ds(row0 + o, s), :],
                send_sem=sem_xs.at[i],
                recv_sem=sem_xr.at[i],
                device_id=x_peer,
                device_id_type=pl.DeviceIdType.MESH,
            )
            r.start()
            rdmas_x.append(r)

        for i in range(C):
            rdmas_y[i].wait_send()
            rdmas_x[i].wait_send()
            rdmas_x[i].wait_recv()

    return pl.pallas_call(
        body,
        out_shape=jax.ShapeDtypeStruct((M, HALF), jnp.bfloat16),
        in_specs=[pl.BlockSpec(memory_space=pl.ANY)],
        out_specs=pl.BlockSpec(memory_space=pltpu.VMEM),
        scratch_shapes=[
            pltpu.VMEM((HALF, N), jnp.float32),
            pltpu.VMEM((HALF, HALF), jnp.bfloat16),
            pltpu.VMEM((HALF, HALF), jnp.bfloat16),
            pltpu.VMEM((HALF, HALF), jnp.bfloat16),
            pltpu.SemaphoreType.DMA((C,)),
            pltpu.SemaphoreType.DMA((C,)),
            pltpu.SemaphoreType.DMA((C,)),
            pltpu.SemaphoreType.DMA((C,)),
            pltpu.SemaphoreType.DMA((C,)),
        ],
        compiler_params=pltpu.CompilerParams(collective_id=0),
    )(x)
